# baseline (speedup 1.0000x reference)
import functools

import jax
import jax.numpy as jnp
from jax import lax
from jax.experimental import pallas as pl
from jax.experimental.pallas import tpu as pltpu
from jax.experimental.pallas import tpu_sc as plsc

D = 128
LANES = 16
COLB = D // LANES


def _make_sc_kernel(b, t, dp, vocab):
    info = plsc.get_sparse_core_info()
    nw = info.num_cores * info.num_subcores
    bpw = b // nw
    k = t * dp
    kpad = (k + 2 * LANES - 1) // LANES * LANES
    ngrp = kpad // LANES

    mesh = plsc.VectorSubcoreMesh(core_axis_name="c", subcore_axis_name="s")

    @functools.partial(
        pl.kernel,
        mesh=mesh,
        out_type=jax.ShapeDtypeStruct((b * t, dp, D), jnp.float32),
        compiler_params=pltpu.CompilerParams(needs_layout_passes=False),
        scratch_types=[
            pltpu.VMEM((bpw * k,), jnp.int32),
            pltpu.VMEM((bpw * k,), jnp.float32),
            pltpu.VMEM((kpad,), jnp.int32),
            pltpu.VMEM((kpad, D), jnp.float32),
            pltpu.VMEM((k, D), jnp.float32),
            pltpu.SemaphoreType.DMA,
            pltpu.SemaphoreType.DMA,
        ],
    )
    def sc_embed(table_hbm, idx_hbm, num_hbm, out_hbm,
                 idx_all, num_all, idx_st, g_v, o_v, sem, wsem):
        wid = lax.axis_index("s") * info.num_cores + lax.axis_index("c")
        wbase = wid * (bpw * k)
        pltpu.sync_copy(idx_hbm.at[pl.ds(wbase, bpw * k)], idx_all)
        pltpu.sync_copy(num_hbm.at[pl.ds(wbase, bpw * k)], num_all)
        lanes = lax.iota(jnp.int32, LANES)

        def b_body(i, _):
            cbase = i * k
            for g in range(ngrp):
                v = plsc.load_gather(idx_all, [jnp.minimum(lanes + (cbase + g * LANES), bpw * k - 1)])
                idx_st[pl.ds(g * LANES, LANES)] = jnp.clip(v, 0, vocab - 1)
            pltpu.async_copy(table_hbm.at[idx_st], g_v, sem).wait()

            bb = wid * bpw + i

            def t_body(tt, _):
                for d in range(dp):
                    row = tt * dp + d
                    s16 = plsc.load_gather(num_all, [jnp.full((LANES,), cbase + row, jnp.int32)])
                    for kk in range(COLB):
                        blk = g_v[row, pl.ds(kk * LANES, LANES)]
                        o_v[row, pl.ds(kk * LANES, LANES)] = blk * s16
                pltpu.async_copy(o_v.at[pl.ds(tt * dp, dp)], out_hbm.at[bb * t + tt], wsem)
                return 0

            lax.fori_loop(0, t, t_body, 0)

            def drain_body(tt, _):
                pltpu.make_async_copy(o_v.at[pl.ds(tt * dp, dp)], out_hbm.at[bb * t + tt], wsem).wait()
                return 0

            lax.fori_loop(0, t, drain_body, 0)
            return 0

        lax.fori_loop(0, bpw, b_body, 0)

    return sc_embed


def kernel(idx, num, table):
    b, t, dp = idx.shape
    n = b * t * dp
    vocab = table.shape[0]
    idx_flat = idx.reshape(n).astype(jnp.int32)
    num_flat = num.reshape(n)
    out = _make_sc_kernel(b, t, dp, vocab)(table, idx_flat, num_flat)
    return out.reshape(b, t, dp, D)
